# baseline (speedup 1.0000x reference)
import functools

import jax
import jax.numpy as jnp
from jax import lax
from jax.experimental import pallas as pl
from jax.experimental.pallas import tpu as pltpu
from jax.experimental.pallas import tpu_sc as plsc

N = 10000
E = 320000
D = 128
H = 128

NC = 2
NS = 16
NW = NC * NS
ROWS = D // NS
PAIRS = ROWS // 2

EC = E // NC
CH = 3200
NCHUNK = EC // CH
GRP = CH // 16
CNT_PER = E // NW
CCH = 2000
NCCH = CNT_PER // CCH


def _unpack_edges(w16):
    halves = plsc.bitcast(w16, jnp.int16)
    return plsc.unpack(halves, format=plsc.PackFormat.INTERLEAVED,
                       preferred_element_type=jnp.int32)


def _sc_body(with_counts, y_hbm, ep_hbm, s_out, *rest):
    if with_counts:
        cnt_out = rest[0]
        rest = rest[1:]
    table_v = rest[0:PAIRS]
    acc_v = rest[PAIRS:PAIRS + ROWS]
    e_b = rest[PAIRS + ROWS:PAIRS + ROWS + 2]
    sem_t = rest[PAIRS + ROWS + 2]
    sem_e = rest[PAIRS + ROWS + 3]

    eh = lax.axis_index("c")
    rg = lax.axis_index("s")
    wid = rg * NC + eh

    z16f = jnp.zeros((16,), jnp.float32)
    ones16 = jnp.ones((16,), jnp.float32)

    for p in range(PAIRS):
        pltpu.async_copy(y_hbm.at[pl.ds((PAIRS * rg + p) * N, N)],
                         table_v[p], sem_t)

    if with_counts:
        @pl.loop(0, N // 16, unroll=8)
        def _(i):
            acc_v[0][pl.ds(i * 16, 16)] = z16f

        @pl.loop(0, NCCH)
        def _(k):
            pltpu.sync_copy(
                ep_hbm.at[pl.ds(wid * CNT_PER + k * CCH, CCH)],
                e_b[0].at[pl.ds(0, CCH)],
            )

            @pl.loop(0, CCH // 16, unroll=8)
            def _(g):
                _, d16 = _unpack_edges(e_b[0][pl.ds(g * 16, 16)])
                plsc.addupdate_scatter(acc_v[0], [d16], ones16)

        pltpu.sync_copy(acc_v[0], cnt_out.at[pl.ds(wid * N, N)])

    ebase = eh * EC
    for b in range(2):
        pltpu.async_copy(ep_hbm.at[pl.ds(ebase + b * CH, CH)], e_b[b], sem_e)

    for c in range(ROWS):
        @pl.loop(0, N // 16, unroll=8)
        def _(i, c=c):
            acc_v[c][pl.ds(i * 16, 16)] = z16f

    for p in range(PAIRS):
        pltpu.make_async_copy(y_hbm.at[pl.ds(0, N)], table_v[p], sem_t).wait()

    @pl.loop(0, NCHUNK, step=2)
    def _(k):
        for b in range(2):
            ev = e_b[b]
            pltpu.make_async_copy(ep_hbm.at[pl.ds(0, CH)], ev, sem_e).wait()

            def gather_unpack(s16):
                vals = []
                for p in range(PAIRS):
                    w = plsc.load_gather(table_v[p], [s16])
                    pair = plsc.bitcast(w, jnp.bfloat16)
                    lo, hi = plsc.unpack(pair,
                                         format=plsc.PackFormat.INTERLEAVED)
                    vals += [lo, hi]
                return vals

            s0, d0 = _unpack_edges(ev[pl.ds(0, 16)])
            v0 = gather_unpack(s0)

            @plsc.parallel_loop(1, GRP, unroll=8, carry=(d0, *v0))
            def _body(g, carry, ev=ev):
                d_prev = carry[0]
                s16, d16 = _unpack_edges(ev[pl.ds(g * 16, 16)])
                for c in range(ROWS):
                    plsc.addupdate_scatter(acc_v[c], [d_prev], carry[1 + c])
                vals = gather_unpack(s16)
                return (d16, *vals)

            for c in range(ROWS):
                plsc.addupdate_scatter(acc_v[c], [_body[0]], _body[1 + c])

            nxt = k + 2 + b

            @pl.when(nxt < NCHUNK)
            def _(ev=ev, nxt=nxt):
                pltpu.async_copy(ep_hbm.at[pl.ds(ebase + nxt * CH, CH)],
                                 ev, sem_e)

    for c in range(ROWS):
        pltpu.sync_copy(
            acc_v[c], s_out.at[pl.ds((eh * D + ROWS * rg + c) * N, N)])


def _make_sc_kernel(with_counts):
    outs = [jax.ShapeDtypeStruct((NC * D * N,), jnp.float32)]
    scratch = (
        [pltpu.VMEM((N,), jnp.int32) for _ in range(PAIRS)]
        + [pltpu.VMEM((N,), jnp.float32) for _ in range(ROWS)]
        + [pltpu.VMEM((CH,), jnp.int32) for _ in range(2)]
        + [pltpu.SemaphoreType.DMA, pltpu.SemaphoreType.DMA]
    )
    if with_counts:
        outs.append(jax.ShapeDtypeStruct((NW * N,), jnp.float32))
    mesh = plsc.VectorSubcoreMesh(core_axis_name="c", subcore_axis_name="s")
    return pl.kernel(
        functools.partial(_sc_body, with_counts),
        out_type=tuple(outs) if with_counts else outs[0],
        mesh=mesh,
        scratch_types=scratch,
        compiler_params=pltpu.CompilerParams(needs_layout_passes=False),
        name="sage_seg_sum" + ("_cnt" if with_counts else ""),
    )


_sc_sum_cnt = _make_sc_kernel(True)
_sc_sum = _make_sc_kernel(False)


def _pack_pairs(y_lo, y_hi):
    lo = lax.bitcast_convert_type(
        y_lo.astype(jnp.bfloat16), jnp.uint16).astype(jnp.uint32)
    hi = lax.bitcast_convert_type(
        y_hi.astype(jnp.bfloat16), jnp.uint16).astype(jnp.uint32)
    return lax.bitcast_convert_type(lo | (hi << 16), jnp.int32)


def _tc_pre_body(xt_ref, wlo_ref, whi_ref, e_ref, y1p_ref, ep_ref):
    xt = xt_ref[...]
    y_lo = lax.dot_general(wlo_ref[...], xt, (((0,), (0,)), ((), ())),
                           preferred_element_type=jnp.float32)
    y_hi = lax.dot_general(whi_ref[...], xt, (((0,), (0,)), ((), ())),
                           preferred_element_type=jnp.float32)
    y1p_ref[...] = _pack_pairs(y_lo, y_hi)
    eu = lax.bitcast_convert_type(e_ref[...], jnp.uint32)
    ep_ref[...] = lax.bitcast_convert_type(
        eu[0:1, :] | (eu[1:2, :] << 16), jnp.int32)


def _tc_mid_body(s1_ref, cnt_ref, xt_ref, wr1_ref, wl2lo_ref, wl2hi_ref,
                 b1_ref, y2p_ref, h_ref):
    c = jnp.sum(cnt_ref[...], axis=0, keepdims=True)
    cmax = jnp.maximum(c, 1.0)
    s1 = s1_ref[0:H, :] + s1_ref[H:2 * H, :]
    m1t = s1 / cmax
    xr = lax.dot_general(wr1_ref[...], xt_ref[...], (((0,), (0,)), ((), ())),
                         preferred_element_type=jnp.float32)
    h = jnp.maximum(m1t + xr + b1_ref[...], 0.0)
    y2_lo = lax.dot_general(wl2lo_ref[...], h, (((0,), (0,)), ((), ())),
                            preferred_element_type=jnp.float32)
    y2_hi = lax.dot_general(wl2hi_ref[...], h, (((0,), (0,)), ((), ())),
                            preferred_element_type=jnp.float32)
    y2p_ref[...] = _pack_pairs(y2_lo, y2_hi)
    h_ref[...] = h


def _tc_hr2_body(h_ref, wr2_ref, b2_ref, hr2t_ref):
    hr2t_ref[...] = lax.dot_general(
        wr2_ref[...], h_ref[...], (((0,), (0,)), ((), ())),
        preferred_element_type=jnp.float32) + b2_ref[...]


def _tc_dec_body(s2_ref, cnt_ref, hr2_ref, dw1_ref, db1_ref, dw2t_ref,
                 db2_ref, out_ref):
    c = jnp.sum(cnt_ref[...], axis=0, keepdims=True)
    cmax = jnp.maximum(c, 1.0)
    s2 = s2_ref[0:H, :] + s2_ref[H:2 * H, :]
    h2 = s2 / cmax + hr2_ref[...]
    z = jnp.maximum(
        lax.dot_general(dw1_ref[...], h2, (((0,), (0,)), ((), ())),
                        preferred_element_type=jnp.float32) + db1_ref[...], 0.0)
    out_ref[...] = lax.dot_general(dw2t_ref[...], z, (((1,), (0,)), ((), ())),
                                   preferred_element_type=jnp.float32) + db2_ref[...]


_tc_pre = pl.pallas_call(
    _tc_pre_body,
    out_shape=[jax.ShapeDtypeStruct((H // 2, N), jnp.int32),
               jax.ShapeDtypeStruct((1, E), jnp.int32)],
)

_tc_mid = pl.pallas_call(
    _tc_mid_body,
    out_shape=[jax.ShapeDtypeStruct((H // 2, N), jnp.int32),
               jax.ShapeDtypeStruct((H, N), jnp.float32)],
)

_tc_hr2 = pl.pallas_call(
    _tc_hr2_body,
    out_shape=jax.ShapeDtypeStruct((H, N), jnp.float32),
)

_tc_dec = pl.pallas_call(
    _tc_dec_body,
    out_shape=jax.ShapeDtypeStruct((1, N), jnp.float32),
)


def kernel(x, edge_index, W_l1, W_r1, b1, W_l2, W_r2, b2, dec_w1, dec_b1,
           dec_w2, dec_b2):
    xt = x.T
    y1p, ep = _tc_pre(xt, W_l1[:, 0::2], W_l1[:, 1::2], edge_index)
    ep = ep.reshape(-1)
    s1t, cnt = _sc_sum_cnt(y1p.reshape(-1), ep)
    s1t = s1t.reshape(NC * H, N)
    cnt = cnt.reshape(NW, N)
    y2p, h = _tc_mid(s1t, cnt, xt, W_r1, W_l2[:, 0::2], W_l2[:, 1::2],
                     b1.reshape(H, 1))
    s2t = _sc_sum(y2p.reshape(-1), ep).reshape(NC * H, N)
    hr2t = _tc_hr2(h, W_r2, b2.reshape(H, 1))
    out = _tc_dec(s2t, cnt, hr2t, dec_w1, dec_b1.reshape(H, 1),
                  dec_w2.T, dec_b2.reshape(1, 1))
    return out.reshape(-1)


# speedup vs baseline: 1.0019x; 1.0019x over previous
import functools

import jax
import jax.numpy as jnp
from jax import lax
from jax.experimental import pallas as pl
from jax.experimental.pallas import tpu as pltpu
from jax.experimental.pallas import tpu_sc as plsc

N = 10000
E = 320000
D = 128
H = 128

NC = 2
NS = 16
NW = NC * NS
ROWS = D // NS
PAIRS = ROWS // 2

EC = E // NC
CH = 3200
NCHUNK = EC // CH
GRP = CH // 16
CNT_PER = E // NW
CCH = 2000
NCCH = CNT_PER // CCH


def _unpack_edges(w16):
    halves = plsc.bitcast(w16, jnp.int16)
    return plsc.unpack(halves, format=plsc.PackFormat.INTERLEAVED,
                       preferred_element_type=jnp.int32)


def _sc_body(with_counts, y_hbm, ep_hbm, s_out, *rest):
    if with_counts:
        cnt_out = rest[0]
        rest = rest[1:]
    table_v = rest[0:PAIRS]
    acc_v = rest[PAIRS:PAIRS + ROWS]
    e_b = rest[PAIRS + ROWS:PAIRS + ROWS + 2]
    sem_t = rest[PAIRS + ROWS + 2]
    sem_e = rest[PAIRS + ROWS + 3]

    eh = lax.axis_index("c")
    rg = lax.axis_index("s")
    wid = rg * NC + eh

    z16f = jnp.zeros((16,), jnp.float32)
    ones16 = jnp.ones((16,), jnp.float32)

    for p in range(PAIRS):
        pltpu.async_copy(y_hbm.at[pl.ds((PAIRS * rg + p) * N, N)],
                         table_v[p], sem_t)

    if with_counts:
        @pl.loop(0, N // 16, unroll=8)
        def _(i):
            acc_v[0][pl.ds(i * 16, 16)] = z16f

        @pl.loop(0, NCCH)
        def _(k):
            pltpu.sync_copy(
                ep_hbm.at[pl.ds(wid * CNT_PER + k * CCH, CCH)],
                e_b[0].at[pl.ds(0, CCH)],
            )

            @pl.loop(0, CCH // 16, unroll=8)
            def _(g):
                _, d16 = _unpack_edges(e_b[0][pl.ds(g * 16, 16)])
                plsc.addupdate_scatter(acc_v[0], [d16], ones16)

        pltpu.sync_copy(acc_v[0], cnt_out.at[pl.ds(wid * N, N)])

    ebase = eh * EC
    for b in range(2):
        pltpu.async_copy(ep_hbm.at[pl.ds(ebase + b * CH, CH)], e_b[b], sem_e)

    for c in range(ROWS):
        @pl.loop(0, N // 16, unroll=8)
        def _(i, c=c):
            acc_v[c][pl.ds(i * 16, 16)] = z16f

    for p in range(PAIRS):
        pltpu.make_async_copy(y_hbm.at[pl.ds(0, N)], table_v[p], sem_t).wait()

    @pl.loop(0, NCHUNK, step=2)
    def _(k):
        for b in range(2):
            ev = e_b[b]
            pltpu.make_async_copy(ep_hbm.at[pl.ds(0, CH)], ev, sem_e).wait()

            def gather_unpack(s16):
                vals = []
                for p in range(PAIRS):
                    w = plsc.load_gather(table_v[p], [s16])
                    pair = plsc.bitcast(w, jnp.bfloat16)
                    lo, hi = plsc.unpack(pair,
                                         format=plsc.PackFormat.INTERLEAVED)
                    vals += [lo, hi]
                return vals

            s0, d0 = _unpack_edges(ev[pl.ds(0, 16)])
            v0 = gather_unpack(s0)

            @plsc.parallel_loop(1, GRP, unroll=8, carry=(d0, *v0))
            def _body(g, carry, ev=ev):
                d_prev = carry[0]
                s16, d16 = _unpack_edges(ev[pl.ds(g * 16, 16)])
                for c in range(ROWS):
                    plsc.addupdate_scatter(acc_v[c], [d_prev], carry[1 + c])
                vals = gather_unpack(s16)
                return (d16, *vals)

            for c in range(ROWS):
                plsc.addupdate_scatter(acc_v[c], [_body[0]], _body[1 + c])

            nxt = k + 2 + b

            @pl.when(nxt < NCHUNK)
            def _(ev=ev, nxt=nxt):
                pltpu.async_copy(ep_hbm.at[pl.ds(ebase + nxt * CH, CH)],
                                 ev, sem_e)

    for c in range(ROWS):
        pltpu.sync_copy(
            acc_v[c], s_out.at[pl.ds((eh * D + ROWS * rg + c) * N, N)])


def _make_sc_kernel(with_counts):
    outs = [jax.ShapeDtypeStruct((NC * D * N,), jnp.float32)]
    scratch = (
        [pltpu.VMEM((N,), jnp.int32) for _ in range(PAIRS)]
        + [pltpu.VMEM((N,), jnp.float32) for _ in range(ROWS)]
        + [pltpu.VMEM((CH,), jnp.int32) for _ in range(2)]
        + [pltpu.SemaphoreType.DMA, pltpu.SemaphoreType.DMA]
    )
    if with_counts:
        outs.append(jax.ShapeDtypeStruct((NW * N,), jnp.float32))
    mesh = plsc.VectorSubcoreMesh(core_axis_name="c", subcore_axis_name="s")
    return pl.kernel(
        functools.partial(_sc_body, with_counts),
        out_type=tuple(outs) if with_counts else outs[0],
        mesh=mesh,
        scratch_types=scratch,
        compiler_params=pltpu.CompilerParams(needs_layout_passes=False),
        name="sage_seg_sum" + ("_cnt" if with_counts else ""),
    )


_sc_sum_cnt = _make_sc_kernel(True)
_sc_sum = _make_sc_kernel(False)


def _pack_pairs(y_lo, y_hi):
    lo = lax.bitcast_convert_type(
        y_lo.astype(jnp.bfloat16), jnp.uint16).astype(jnp.uint32)
    hi = lax.bitcast_convert_type(
        y_hi.astype(jnp.bfloat16), jnp.uint16).astype(jnp.uint32)
    return lax.bitcast_convert_type(lo | (hi << 16), jnp.int32)


def _tc_pre_body(xt_ref, wlo_ref, whi_ref, e_ref, y1p_ref, ep_ref):
    xt = xt_ref[...]
    y_lo = lax.dot_general(wlo_ref[...], xt, (((0,), (0,)), ((), ())),
                           preferred_element_type=jnp.float32)
    y_hi = lax.dot_general(whi_ref[...], xt, (((0,), (0,)), ((), ())),
                           preferred_element_type=jnp.float32)
    y1p_ref[...] = _pack_pairs(y_lo, y_hi)
    eu = lax.bitcast_convert_type(e_ref[...], jnp.uint32)
    ep_ref[...] = lax.bitcast_convert_type(
        eu[0:1, :] | (eu[1:2, :] << 16), jnp.int32)


def _tc_mid_body(s1_ref, cnt_ref, xt_ref, wr1_ref, wl2lo_ref, wl2hi_ref,
                 wr2_ref, b1_ref, b2_ref, y2p_ref, hr2t_ref):
    c = jnp.sum(cnt_ref[...], axis=0, keepdims=True)
    cmax = jnp.maximum(c, 1.0)
    s1 = s1_ref[0:H, :] + s1_ref[H:2 * H, :]
    m1t = s1 / cmax
    xr = lax.dot_general(wr1_ref[...], xt_ref[...], (((0,), (0,)), ((), ())),
                         preferred_element_type=jnp.float32)
    h = jnp.maximum(m1t + xr + b1_ref[...], 0.0)
    y2_lo = lax.dot_general(wl2lo_ref[...], h, (((0,), (0,)), ((), ())),
                            preferred_element_type=jnp.float32)
    y2_hi = lax.dot_general(wl2hi_ref[...], h, (((0,), (0,)), ((), ())),
                            preferred_element_type=jnp.float32)
    y2p_ref[...] = _pack_pairs(y2_lo, y2_hi)
    hr2t_ref[...] = lax.dot_general(wr2_ref[...], h, (((0,), (0,)), ((), ())),
                                    preferred_element_type=jnp.float32) + b2_ref[...]


def _tc_dec_body(s2_ref, cnt_ref, hr2_ref, dw1_ref, db1_ref, dw2t_ref,
                 db2_ref, out_ref):
    c = jnp.sum(cnt_ref[...], axis=0, keepdims=True)
    cmax = jnp.maximum(c, 1.0)
    s2 = s2_ref[0:H, :] + s2_ref[H:2 * H, :]
    h2 = s2 / cmax + hr2_ref[...]
    z = jnp.maximum(
        lax.dot_general(dw1_ref[...], h2, (((0,), (0,)), ((), ())),
                        preferred_element_type=jnp.float32) + db1_ref[...], 0.0)
    out_ref[...] = lax.dot_general(dw2t_ref[...], z, (((1,), (0,)), ((), ())),
                                   preferred_element_type=jnp.float32) + db2_ref[...]


_tc_pre = pl.pallas_call(
    _tc_pre_body,
    out_shape=[jax.ShapeDtypeStruct((H // 2, N), jnp.int32),
               jax.ShapeDtypeStruct((1, E), jnp.int32)],
)

_tc_mid = pl.pallas_call(
    _tc_mid_body,
    out_shape=[jax.ShapeDtypeStruct((H // 2, N), jnp.int32),
               jax.ShapeDtypeStruct((H, N), jnp.float32)],
)

_tc_dec = pl.pallas_call(
    _tc_dec_body,
    out_shape=jax.ShapeDtypeStruct((1, N), jnp.float32),
)


def kernel(x, edge_index, W_l1, W_r1, b1, W_l2, W_r2, b2, dec_w1, dec_b1,
           dec_w2, dec_b2):
    xt = x.T
    y1p, ep = _tc_pre(xt, W_l1[:, 0::2], W_l1[:, 1::2], edge_index)
    ep = ep.reshape(-1)
    s1t, cnt = _sc_sum_cnt(y1p.reshape(-1), ep)
    s1t = s1t.reshape(NC * H, N)
    cnt = cnt.reshape(NW, N)
    y2p, hr2t = _tc_mid(s1t, cnt, xt, W_r1, W_l2[:, 0::2], W_l2[:, 1::2],
                        W_r2, b1.reshape(H, 1), b2.reshape(H, 1))
    s2t = _sc_sum(y2p.reshape(-1), ep).reshape(NC * H, N)
    out = _tc_dec(s2t, cnt, hr2t, dec_w1, dec_b1.reshape(H, 1),
                  dec_w2.T, dec_b2.reshape(1, 1))
    return out.reshape(-1)
